# baseline (speedup 1.0000x reference)
import functools

import jax
import jax.numpy as jnp
from jax import lax
from jax.experimental import pallas as pl
from jax.experimental.pallas import tpu as pltpu
from jax.experimental.pallas import tpu_sc as plsc

NC = 2
NS = 16
NW = NC * NS
LANES = 16


def _transpose16(vs, lane):
    for s in (1, 2, 4, 8):
        pm = lane ^ s
        mk = (lane & s) == 0
        nv = list(vs)
        for i in range(16):
            if i & s == 0:
                pr = i | s
                lo, hi = vs[i], vs[pr]
                nv[i] = jnp.where(mk, lo,
                                  hi.at[pm].get(mode="promise_in_bounds"))
                nv[pr] = jnp.where(mk,
                                   lo.at[pm].get(mode="promise_in_bounds"), hi)
        vs = nv
    return vs


@jax.jit
def _sc_format_table(tok_t, tail128):
    D, V = tok_t.shape
    npan = V // 128
    tail = V - npan * 128
    per_w = (npan + NW - 1) // NW

    mesh = plsc.VectorSubcoreMesh(core_axis_name="c", subcore_axis_name="s")

    @functools.partial(
        pl.kernel,
        out_type=jax.ShapeDtypeStruct((V, 128), jnp.float32),
        mesh=mesh,
        compiler_params=pltpu.CompilerParams(use_tc_tiling_on_sc=True,
                                             needs_layout_passes=False),
        scratch_types=[
            pltpu.VMEM((D, 128), jnp.float32),
            pltpu.VMEM((D, 128), jnp.float32),
            pltpu.VMEM((D, 128), jnp.float32),
            pltpu.VMEM((128, 128), jnp.float32),
            pltpu.VMEM((128, 128), jnp.float32),
            pltpu.VMEM((128, 128), jnp.float32),
            pltpu.SemaphoreType.DMA,
            pltpu.SemaphoreType.DMA,
            pltpu.SemaphoreType.DMA,
            pltpu.SemaphoreType.DMA,
            pltpu.SemaphoreType.DMA,
            pltpu.SemaphoreType.DMA,
        ],
    )
    def body(tok_hbm, tail_hbm, out_hbm, src0, src1, src2, dst0, dst1, dst2,
             si0, si1, si2, so0, so1, so2):
        c = lax.axis_index("c")
        s = lax.axis_index("s")
        wid = s * NC + c
        p0 = wid * per_w
        pn = jnp.minimum(per_w, npan - p0)
        srcs, dsts = (src0, src1, src2), (dst0, dst1, dst2)
        sis, sos = (si0, si1, si2), (so0, so1, so2)
        lane = lax.iota(jnp.int32, LANES)

        def col0_of(m):
            return pl.multiple_of((p0 + m) * 128, 128)

        def fire_in(m, buf):
            pltpu.async_copy(tok_hbm.at[:, pl.ds(col0_of(m), 128)],
                             srcs[buf], sis[buf])

        for k in range(2):
            @pl.when(k < pn)
            def _():
                fire_in(k, k)

        def do_panel(m, buf):
            @pl.when(m < pn)
            def _():
                pltpu.make_async_copy(
                    tok_hbm.at[:, pl.ds(col0_of(m), 128)], srcs[buf],
                    sis[buf]).wait()

                @pl.when(m + 2 < pn)
                def _():
                    fire_in(m + 2, (buf + 2) % 3)

                @pl.when(m >= 3)
                def _():
                    pltpu.make_async_copy(
                        dsts[buf], out_hbm.at[pl.ds(col0_of(m - 3), 128)],
                        sos[buf]).wait()

                sb, db = srcs[buf], dsts[buf]
                for q in range(D // LANES):
                    for cb in range(128 // LANES):
                        vs = [sb[q * LANES + i, pl.ds(cb * LANES, LANES)]
                              for i in range(LANES)]
                        vs = _transpose16(vs, lane)
                        for i in range(LANES):
                            db[cb * LANES + i, pl.ds(q * LANES, LANES)] = vs[i]

                pltpu.async_copy(db, out_hbm.at[pl.ds(col0_of(m), 128)],
                                 sos[buf])

        def step3(mm, carry):
            for k in range(3):
                do_panel(mm * 3 + k, k)
            return carry

        lax.fori_loop(0, (per_w + 2) // 3, step3, 0)

        for buf in range(3):
            m = pn - 1 - lax.rem(pn - 1 - buf + 3, 3)

            @pl.when(m >= 0)
            def _():
                pltpu.make_async_copy(
                    dsts[buf], out_hbm.at[pl.ds(col0_of(m), 128)],
                    sos[buf]).wait()

        if tail:
            @pl.when(wid == NW - 1)
            def _():
                pltpu.sync_copy(tail_hbm, src0)
                pltpu.sync_copy(src0.at[pl.ds(0, tail)],
                                out_hbm.at[pl.ds(npan * 128, tail)])

    return body(tok_t, tail128)


@functools.partial(jax.jit, static_argnums=(3, 4))
def _sc_embed(idx_t, tok128, pos_table, blk, D):
    L, B = idx_t.shape
    V, DP = tok128.shape

    mesh = plsc.VectorSubcoreMesh(core_axis_name="c", subcore_axis_name="s")

    @functools.partial(
        pl.kernel,
        out_type=jax.ShapeDtypeStruct((L, D, B), jnp.float32),
        mesh=mesh,
        compiler_params=pltpu.CompilerParams(use_tc_tiling_on_sc=True,
                                             needs_layout_passes=False),
        scratch_types=[
            pltpu.VMEM((L, blk), jnp.int32),
            pltpu.VMEM((L, D), jnp.float32),
            pltpu.VMEM((blk, DP), jnp.float32),
            pltpu.VMEM((blk, DP), jnp.float32),
            pltpu.VMEM((D, blk), jnp.float32),
            pltpu.VMEM((D, blk), jnp.float32),
            pltpu.SemaphoreType.DMA,
            pltpu.SemaphoreType.DMA,
            pltpu.SemaphoreType.DMA,
            pltpu.SemaphoreType.DMA,
        ],
    )
    def body(idx_hbm, tok_hbm, pos_hbm, out_hbm,
             idx_v, pos_v, rows0, rows1, tr0, tr1,
             sg0, sg1, so0, so1):
        c = lax.axis_index("c")
        s = lax.axis_index("s")
        wid = s * NC + c
        b0 = wid * blk

        pltpu.sync_copy(idx_hbm.at[:, pl.ds(b0, blk)], idx_v)
        pltpu.sync_copy(pos_hbm, pos_v)

        rows = (rows0, rows1)
        trs = (tr0, tr1)
        sgs = (sg0, sg1)
        sos = (so0, so1)

        def fire_gather(p, buf):
            pltpu.async_copy(tok_hbm.at[idx_v.at[p]], rows[buf], sgs[buf])

        fire_gather(0, 0)

        def do_pos(p, buf):
            pltpu.make_async_copy(tok_hbm.at[idx_v.at[p]], rows[buf],
                                  sgs[buf]).wait()

            @pl.when(p + 1 < L)
            def _():
                fire_gather(p + 1, 1 - buf)

            @pl.when(p >= 2)
            def _():
                pltpu.make_async_copy(
                    trs[buf], out_hbm.at[p - 2, :, pl.ds(b0, blk)],
                    sos[buf]).wait()

            rbuf = rows[buf]
            tbuf = trs[buf]
            nq = D // LANES
            pvs = tuple(pos_v[p, pl.ds(q * LANES, LANES)] for q in range(nq))
            lane = lax.iota(jnp.int32, LANES)
            perms = {s: lane ^ s for s in (1, 2, 4, 8)}
            masks = {s: (lane & s) == 0 for s in (1, 2, 4, 8)}

            def do_bchunk(cb, carry):
                bb = cb * LANES
                for q in range(nq):
                    vs = [rbuf[bb + i, pl.ds(q * LANES, LANES)] + carry[q]
                          for i in range(LANES)]
                    for s in (1, 2, 4, 8):
                        pm, mk = perms[s], masks[s]
                        nv = list(vs)
                        for i in range(LANES):
                            if i & s == 0:
                                pr = i | s
                                lo, hi = vs[i], vs[pr]
                                nv[i] = jnp.where(
                                    mk, lo, hi.at[pm].get(mode="promise_in_bounds"))
                                nv[pr] = jnp.where(
                                    mk, lo.at[pm].get(mode="promise_in_bounds"), hi)
                        vs = nv
                    for i in range(LANES):
                        tbuf[q * LANES + i, pl.ds(bb, LANES)] = vs[i]
                return carry

            lax.fori_loop(0, blk // LANES, do_bchunk, pvs)

            pltpu.async_copy(tbuf, out_hbm.at[p, :, pl.ds(b0, blk)],
                             sos[buf])

        def step2(pp, carry):
            do_pos(pp * 2, 0)
            do_pos(pp * 2 + 1, 1)
            return carry

        lax.fori_loop(0, L // 2, step2, 0)

        for buf in range(2):
            p = L - 2 + buf
            pltpu.make_async_copy(trs[buf], out_hbm.at[p, :, pl.ds(b0, blk)],
                                  sos[buf]).wait()

    return body(idx_t, tok128, pos_table)


def kernel(inputs, token_table, pos_table):
    B, L = inputs.shape
    V, D = token_table.shape
    idx_t = jnp.transpose(inputs).astype(jnp.int32)
    ntail = V % 128
    tail128 = jnp.pad(token_table[V - ntail:], ((0, 64 - ntail), (0, 128 - D)))
    tok128 = _sc_format_table(jnp.transpose(token_table), tail128)
    out_phys = _sc_embed(idx_t, tok128, pos_table, B // NW, D)
    return jnp.transpose(out_phys, (2, 0, 1))


# speedup vs baseline: 1.0389x; 1.0389x over previous
import functools

import jax
import jax.numpy as jnp
from jax import lax
from jax.experimental import pallas as pl
from jax.experimental.pallas import tpu as pltpu
from jax.experimental.pallas import tpu_sc as plsc

NC = 2
NS = 16
NW = NC * NS
LANES = 16


def _transpose16(vs, lane):
    for s in (1, 2, 4, 8):
        pm = lane ^ s
        mk = (lane & s) == 0
        nv = list(vs)
        for i in range(16):
            if i & s == 0:
                pr = i | s
                lo, hi = vs[i], vs[pr]
                nv[i] = jnp.where(mk, lo,
                                  hi.at[pm].get(mode="promise_in_bounds"))
                nv[pr] = jnp.where(mk,
                                   lo.at[pm].get(mode="promise_in_bounds"), hi)
        vs = nv
    return vs


@functools.partial(jax.jit, static_argnums=(4, 5))
def _sc_embed(idxh_t, off_t, tok2, pos2, blk, D):
    L, B = idxh_t.shape
    V2, DP = tok2.shape

    mesh = plsc.VectorSubcoreMesh(core_axis_name="c", subcore_axis_name="s")

    @functools.partial(
        pl.kernel,
        out_type=jax.ShapeDtypeStruct((L, D, B), jnp.float32),
        mesh=mesh,
        compiler_params=pltpu.CompilerParams(use_tc_tiling_on_sc=True,
                                             needs_layout_passes=False),
        scratch_types=[
            pltpu.VMEM((L, blk), jnp.int32),
            pltpu.VMEM((L, blk), jnp.int32),
            pltpu.VMEM((L // 2, 2 * D), jnp.float32),
            pltpu.VMEM((blk, DP), jnp.float32),
            pltpu.VMEM((blk, DP), jnp.float32),
            pltpu.VMEM((D, blk), jnp.float32),
            pltpu.VMEM((D, blk), jnp.float32),
            pltpu.SemaphoreType.DMA,
            pltpu.SemaphoreType.DMA,
            pltpu.SemaphoreType.DMA,
            pltpu.SemaphoreType.DMA,
        ],
    )
    def body(idxh_hbm, off_hbm, tok_hbm, pos_hbm, out_hbm,
             idxh_v, off_v, pos_v, rows0, rows1, tr0, tr1,
             sg0, sg1, so0, so1):
        c = lax.axis_index("c")
        s = lax.axis_index("s")
        wid = s * NC + c
        b0 = wid * blk

        pltpu.sync_copy(idxh_hbm.at[:, pl.ds(b0, blk)], idxh_v)
        pltpu.sync_copy(off_hbm.at[:, pl.ds(b0, blk)], off_v)
        pltpu.sync_copy(pos_hbm, pos_v)

        rows = (rows0, rows1)
        trs = (tr0, tr1)
        sgs = (sg0, sg1)
        sos = (so0, so1)

        def fire_gather(p, buf):
            pltpu.async_copy(tok_hbm.at[idxh_v.at[p]], rows[buf], sgs[buf])

        fire_gather(0, 0)

        def do_pos(p, buf):
            pltpu.make_async_copy(tok_hbm.at[idxh_v.at[p]], rows[buf],
                                  sgs[buf]).wait()

            @pl.when(p + 1 < L)
            def _():
                fire_gather(p + 1, 1 - buf)

            @pl.when(p >= 2)
            def _():
                pltpu.make_async_copy(
                    trs[buf], out_hbm.at[p - 2, :, pl.ds(b0, blk)],
                    sos[buf]).wait()

            rbuf = rows[buf]
            tbuf = trs[buf]
            nq = D // LANES
            ppos = (p % 2) * D
            pvs = tuple(pos_v[p // 2, pl.ds(ppos + q * LANES, LANES)]
                        for q in range(nq))
            lane = lax.iota(jnp.int32, LANES)
            perms = {s: lane ^ s for s in (1, 2, 4, 8)}
            masks = {s: (lane & s) == 0 for s in (1, 2, 4, 8)}

            def do_bchunk(cb, carry):
                bb = cb * LANES
                offs_vec = off_v[p, pl.ds(bb, LANES)]
                offs = [offs_vec[i] for i in range(LANES)]
                for q in range(nq):
                    vs = [rbuf[bb + i, pl.ds(offs[i] + q * LANES, LANES)]
                          + carry[q] for i in range(LANES)]
                    for s in (1, 2, 4, 8):
                        pm, mk = perms[s], masks[s]
                        nv = list(vs)
                        for i in range(LANES):
                            if i & s == 0:
                                pr = i | s
                                lo, hi = vs[i], vs[pr]
                                nv[i] = jnp.where(
                                    mk, lo, hi.at[pm].get(mode="promise_in_bounds"))
                                nv[pr] = jnp.where(
                                    mk, lo.at[pm].get(mode="promise_in_bounds"), hi)
                        vs = nv
                    for i in range(LANES):
                        tbuf[q * LANES + i, pl.ds(bb, LANES)] = vs[i]
                return carry

            lax.fori_loop(0, blk // LANES, do_bchunk, pvs)

            pltpu.async_copy(tbuf, out_hbm.at[p, :, pl.ds(b0, blk)],
                             sos[buf])

        def step2(pp, carry):
            do_pos(pp * 2, 0)
            do_pos(pp * 2 + 1, 1)
            return carry

        lax.fori_loop(0, L // 2, step2, 0)

        for buf in range(2):
            p = L - 2 + buf
            pltpu.make_async_copy(trs[buf], out_hbm.at[p, :, pl.ds(b0, blk)],
                                  sos[buf]).wait()

    return body(idxh_t, off_t, tok2, pos2)


def kernel(inputs, token_table, pos_table):
    B, L = inputs.shape
    V, D = token_table.shape
    idx_t = jnp.transpose(inputs).astype(jnp.int32)
    idxh_t = idx_t >> 1
    off_t = (idx_t & 1) * D
    tok2 = jnp.reshape(token_table, (V // 2, 2 * D))
    pos2 = jnp.reshape(pos_table, (L // 2, 2 * D))
    out_phys = _sc_embed(idxh_t, off_t, tok2, pos2, B // NW, D)
    return jnp.transpose(out_phys, (2, 0, 1))


# speedup vs baseline: 1.2142x; 1.1687x over previous
import functools

import jax
import jax.numpy as jnp
from jax import lax
from jax.experimental import pallas as pl
from jax.experimental.pallas import tpu as pltpu
from jax.experimental.pallas import tpu_sc as plsc

NC = 2
NS = 16
NW = NC * NS
LANES = 16


def _transpose16(vs, lane):
    for s in (1, 2, 4, 8):
        pm = lane ^ s
        mk = (lane & s) == 0
        nv = list(vs)
        for i in range(16):
            if i & s == 0:
                pr = i | s
                lo, hi = vs[i], vs[pr]
                nv[i] = jnp.where(mk, lo,
                                  hi.at[pm].get(mode="promise_in_bounds"))
                nv[pr] = jnp.where(mk,
                                   lo.at[pm].get(mode="promise_in_bounds"), hi)
        vs = nv
    return vs


@functools.partial(jax.jit, static_argnums=(3, 4))
def _sc_embed(idx_t, tok128, pos2, blk, D):
    L, B = idx_t.shape
    V, DP = tok128.shape

    mesh = plsc.VectorSubcoreMesh(core_axis_name="c", subcore_axis_name="s")

    @functools.partial(
        pl.kernel,
        out_type=jax.ShapeDtypeStruct((L, D, B), jnp.float32),
        mesh=mesh,
        compiler_params=pltpu.CompilerParams(use_tc_tiling_on_sc=True,
                                             needs_layout_passes=False),
        scratch_types=[
            pltpu.VMEM((L, blk), jnp.int32),
            pltpu.VMEM((L // 2, 2 * D), jnp.float32),
            pltpu.VMEM((blk, DP), jnp.float32),
            pltpu.VMEM((blk, DP), jnp.float32),
            pltpu.VMEM((blk, DP), jnp.float32),
            pltpu.VMEM((D, blk), jnp.float32),
            pltpu.VMEM((D, blk), jnp.float32),
            pltpu.VMEM((D, blk), jnp.float32),
            pltpu.SemaphoreType.DMA,
            pltpu.SemaphoreType.DMA,
            pltpu.SemaphoreType.DMA,
            pltpu.SemaphoreType.DMA,
            pltpu.SemaphoreType.DMA,
            pltpu.SemaphoreType.DMA,
        ],
    )
    def body(idx_hbm, tok_hbm, pos_hbm, out_hbm,
             idx_v, pos_v, rows0, rows1, rows2, tr0, tr1, tr2,
             sg0, sg1, sg2, so0, so1, so2):
        c = lax.axis_index("c")
        s = lax.axis_index("s")
        wid = s * NC + c
        b0 = wid * blk

        pltpu.sync_copy(idx_hbm.at[:, pl.ds(b0, blk)], idx_v)
        pltpu.sync_copy(pos_hbm, pos_v)

        rows = (rows0, rows1, rows2)
        trs = (tr0, tr1, tr2)
        sgs = (sg0, sg1, sg2)
        sos = (so0, so1, so2)

        def fire_gather(p, buf):
            pltpu.async_copy(tok_hbm.at[idx_v.at[p]], rows[buf], sgs[buf])

        for k in range(2):
            fire_gather(k, k)

        def do_pos(p, buf):
            pltpu.make_async_copy(tok_hbm.at[idx_v.at[p]], rows[buf],
                                  sgs[buf]).wait()

            @pl.when(p + 2 < L)
            def _():
                fire_gather(p + 2, (buf + 2) % 3)

            @pl.when(p >= 3)
            def _():
                pltpu.make_async_copy(
                    trs[buf], out_hbm.at[p - 3, :, pl.ds(b0, blk)],
                    sos[buf]).wait()

            rbuf = rows[buf]
            tbuf = trs[buf]
            nq = D // LANES
            ppos = (p % 2) * D
            pvs = tuple(pos_v[p // 2, pl.ds(ppos + q * LANES, LANES)]
                        for q in range(nq))
            lane = lax.iota(jnp.int32, LANES)
            perms = {s: lane ^ s for s in (1, 2, 4, 8)}
            masks = {s: (lane & s) == 0 for s in (1, 2, 4, 8)}

            def do_bchunk(cb, carry):
                bb = cb * LANES
                for q in range(nq):
                    vs = [rbuf[bb + i, pl.ds(q * LANES, LANES)]
                          + carry[q] for i in range(LANES)]
                    for s in (1, 2, 4, 8):
                        pm, mk = perms[s], masks[s]
                        nv = list(vs)
                        for i in range(LANES):
                            if i & s == 0:
                                pr = i | s
                                lo, hi = vs[i], vs[pr]
                                nv[i] = jnp.where(
                                    mk, lo, hi.at[pm].get(mode="promise_in_bounds"))
                                nv[pr] = jnp.where(
                                    mk, lo.at[pm].get(mode="promise_in_bounds"), hi)
                        vs = nv
                    for i in range(LANES):
                        tbuf[q * LANES + i, pl.ds(bb, LANES)] = vs[i]
                return carry

            lax.fori_loop(0, blk // LANES, do_bchunk, pvs)

            pltpu.async_copy(tbuf, out_hbm.at[p, :, pl.ds(b0, blk)],
                             sos[buf])

        def step3(pp, carry):
            for k in range(3):
                do_pos(pp * 3 + k, k)
            return carry

        assert L % 3 != 0 or True
        nfull = L // 3
        lax.fori_loop(0, nfull, step3, 0)
        for k in range(L - nfull * 3):
            do_pos(nfull * 3 + k, k)

        for buf in range(3):
            p = L - 1 - (L - 1 - buf) % 3
            pltpu.make_async_copy(trs[buf], out_hbm.at[p, :, pl.ds(b0, blk)],
                                  sos[buf]).wait()

    return body(idx_t, tok128, pos2)


def kernel(inputs, token_table, pos_table):
    B, L = inputs.shape
    V, D = token_table.shape
    idx_t = jnp.transpose(inputs).astype(jnp.int32)
    tok128 = jnp.pad(token_table, ((0, 0), (0, 128 - D)))
    pos2 = jnp.reshape(pos_table, (L // 2, 2 * D))
    out_phys = _sc_embed(idx_t, tok128, pos2, B // NW, D)
    return jnp.transpose(out_phys, (2, 0, 1))


# speedup vs baseline: 1.2166x; 1.0020x over previous
import functools

import jax
import jax.numpy as jnp
from jax import lax
from jax.experimental import pallas as pl
from jax.experimental.pallas import tpu as pltpu
from jax.experimental.pallas import tpu_sc as plsc

NC = 2
NS = 16
NW = NC * NS
LANES = 16


def _transpose16(vs, lane):
    for s in (1, 2, 4, 8):
        pm = lane ^ s
        mk = (lane & s) == 0
        nv = list(vs)
        for i in range(16):
            if i & s == 0:
                pr = i | s
                lo, hi = vs[i], vs[pr]
                nv[i] = jnp.where(mk, lo,
                                  hi.at[pm].get(mode="promise_in_bounds"))
                nv[pr] = jnp.where(mk,
                                   lo.at[pm].get(mode="promise_in_bounds"), hi)
        vs = nv
    return vs


@functools.partial(jax.jit, static_argnums=(3, 4))
def _sc_embed(idx_t, tok128, pos2, blk, D):
    L, B = idx_t.shape
    V, DP = tok128.shape

    mesh = plsc.VectorSubcoreMesh(core_axis_name="c", subcore_axis_name="s")

    @functools.partial(
        pl.kernel,
        out_type=jax.ShapeDtypeStruct((L, D, B), jnp.float32),
        mesh=mesh,
        compiler_params=pltpu.CompilerParams(use_tc_tiling_on_sc=True,
                                             needs_layout_passes=False),
        scratch_types=[
            pltpu.VMEM((L, blk), jnp.int32),
            pltpu.VMEM((L // 2, 2 * D), jnp.float32),
            pltpu.VMEM((blk, DP), jnp.float32),
            pltpu.VMEM((blk, DP), jnp.float32),
            pltpu.VMEM((blk, DP), jnp.float32),
            pltpu.VMEM((D, blk), jnp.float32),
            pltpu.VMEM((D, blk), jnp.float32),
            pltpu.VMEM((D, blk), jnp.float32),
            pltpu.SemaphoreType.DMA,
            pltpu.SemaphoreType.DMA,
            pltpu.SemaphoreType.DMA,
            pltpu.SemaphoreType.DMA,
            pltpu.SemaphoreType.DMA,
            pltpu.SemaphoreType.DMA,
        ],
    )
    def body(idx_hbm, tok_hbm, pos_hbm, out_hbm,
             idx_v, pos_v, rows0, rows1, rows2, tr0, tr1, tr2,
             sg0, sg1, sg2, so0, so1, so2):
        c = lax.axis_index("c")
        s = lax.axis_index("s")
        wid = s * NC + c
        b0 = wid * blk

        pltpu.sync_copy(idx_hbm.at[:, pl.ds(b0, blk)], idx_v)
        pltpu.sync_copy(pos_hbm, pos_v)

        rows = (rows0, rows1, rows2)
        trs = (tr0, tr1, tr2)
        sgs = (sg0, sg1, sg2)
        sos = (so0, so1, so2)

        def fire_gather(p, buf):
            pltpu.async_copy(tok_hbm.at[idx_v.at[p]], rows[buf], sgs[buf])

        for k in range(2):
            fire_gather(k, k)

        def do_pos(p, buf):
            pltpu.make_async_copy(tok_hbm.at[idx_v.at[p]], rows[buf],
                                  sgs[buf]).wait()

            @pl.when(p + 2 < L)
            def _():
                fire_gather(p + 2, (buf + 2) % 3)

            @pl.when(p >= 3)
            def _():
                pltpu.make_async_copy(
                    trs[buf], out_hbm.at[p - 3, :, pl.ds(b0, blk)],
                    sos[buf]).wait()

            rbuf = rows[buf]
            tbuf = trs[buf]
            nq = D // LANES
            ppos = (p % 2) * D
            pvs = tuple(pos_v[p // 2, pl.ds(ppos + q * LANES, LANES)]
                        for q in range(nq))
            lane = lax.iota(jnp.int32, LANES)
            perms = {s: lane ^ s for s in (1, 2, 4, 8)}
            masks = {s: (lane & s) == 0 for s in (1, 2, 4, 8)}

            def do_bchunk(cb, carry):
                bb = cb * LANES
                for q in range(nq):
                    vs = [rbuf[bb + i, pl.ds(q * LANES, LANES)]
                          + carry[q] for i in range(LANES)]
                    for s in (1, 2, 4, 8):
                        pm, mk = perms[s], masks[s]
                        nv = list(vs)
                        for i in range(LANES):
                            if i & s == 0:
                                pr = i | s
                                lo, hi = vs[i], vs[pr]
                                nv[i] = jnp.where(
                                    mk, lo, hi.at[pm].get(mode="promise_in_bounds"))
                                nv[pr] = jnp.where(
                                    mk, lo.at[pm].get(mode="promise_in_bounds"), hi)
                        vs = nv
                    for i in range(LANES):
                        tbuf[q * LANES + i, pl.ds(bb, LANES)] = vs[i]
                return carry

            lax.fori_loop(0, blk // LANES, do_bchunk, pvs)

            pltpu.async_copy(tbuf, out_hbm.at[p, :, pl.ds(b0, blk)],
                             sos[buf])

        def step3(pp, carry):
            for k in range(3):
                do_pos(pp * 3 + k, k)
            return carry

        nfull = L // 3
        lax.fori_loop(0, nfull, step3, 0)
        for k in range(L - nfull * 3):
            do_pos(nfull * 3 + k, k)

        for buf in range(3):
            p = L - 1 - (L - 1 - buf) % 3
            pltpu.make_async_copy(trs[buf], out_hbm.at[p, :, pl.ds(b0, blk)],
                                  sos[buf]).wait()

    return body(idx_t, tok128, pos2)


def kernel(inputs, token_table, pos_table):
    B, L = inputs.shape
    V, D = token_table.shape
    idx_t = jnp.transpose(inputs).astype(jnp.int32)
    tok128 = jnp.pad(token_table, ((0, 0), (0, 128 - D)))
    pos2 = jnp.reshape(pos_table, (L // 2, 2 * D))
    out_phys = _sc_embed(idx_t, tok128, pos2, B // NW, D)
    return jnp.transpose(out_phys, (2, 0, 1))
